# baseline (speedup 1.0000x reference)
import functools
import jax
import jax.numpy as jnp
from jax import lax
from jax.experimental import pallas as pl
from jax.experimental.pallas import tpu as pltpu
from jax.experimental.pallas import tpu_sc as plsc

NN = 10000
EE = 160000
DIN = 256
DH = 128
NGRAPH = 64
NCLS = 16
NC, NS = 2, 16
NW = NC * NS

HIST = 10112
DEG_ROWS = 40
AGG_EPT = EE // NS
CH = 125
NCHUNK = AGG_EPT // CH

NP = 10112
RPT = NP // NS
BLK = 1264
NBLK = NP // BLK

_HI = jax.lax.Precision.DEFAULT


def _deg_body(er_hbm, out_hbm, dst_v, hist_v):
    c = lax.axis_index("c")
    s = lax.axis_index("s")
    wid = s * NC + c
    t = wid // 2
    half = wid % 2
    pltpu.sync_copy(er_hbm.at[1].at[t].at[pl.ds(half * DEG_ROWS, DEG_ROWS)],
                    dst_v)

    def zero(i, _):
        hist_v[pl.ds(i * 16, 16)] = jnp.zeros((16,), jnp.float32)
        return 0

    lax.fori_loop(0, HIST // 16, zero, 0)
    ones = jnp.ones((16,), jnp.float32)
    tail = lax.broadcasted_iota(jnp.int32, (16,), 0) >= 3

    def row(r, _):
        for q in range(7):
            idx = dst_v[r, pl.ds(q * 16, 16)]
            plsc.addupdate_scatter(hist_v, [idx], ones)
        idx = dst_v[r, pl.ds(CH - 16, 16)]
        plsc.addupdate_scatter(hist_v, [idx], ones, mask=tail)
        return 0

    lax.fori_loop(0, DEG_ROWS, row, 0)
    pltpu.sync_copy(hist_v, out_hbm.at[wid])


@functools.cache
def _deg_call():
    return pl.kernel(
        _deg_body,
        out_type=jax.ShapeDtypeStruct((NW, HIST), jnp.float32),
        mesh=plsc.VectorSubcoreMesh(core_axis_name="c", subcore_axis_name="s"),
        scratch_types=[
            pltpu.VMEM((DEG_ROWS, CH), jnp.int32),
            pltpu.VMEM((HIST,), jnp.float32),
        ],
        compiler_params=pltpu.CompilerParams(needs_layout_passes=False),
    )


NBUF = 2
IG = 8
NGROUP = NCHUNK // IG


def _agg_body(hp_hbm, er_hbm, out_hbm, srcb, dstb, rows_v, acc_sh,
              *sems):
    gsems, ssems, isems = sems[0:2], sems[2:4], sems[4:6]
    c = lax.axis_index("c")
    t = lax.axis_index("s")
    hp = hp_hbm.at[c]
    out = out_hbm.at[c]
    srch = er_hbm.at[0].at[t]
    dsth = er_hbm.at[1].at[t]
    pltpu.sync_copy(hp.at[pl.ds(t * RPT, RPT)],
                    acc_sh.at[pl.ds(t * RPT, RPT)])
    for p in range(2):
        pltpu.async_copy(srch.at[pl.ds(p * IG, IG)], srcb.at[p], isems[p])
        pltpu.async_copy(dsth.at[pl.ds(p * IG, IG)], dstb.at[p], isems[p])
    plsc.subcore_barrier()
    pltpu.make_async_copy(srch.at[pl.ds(0, IG)], srcb.at[0], isems[0]).wait()
    pltpu.make_async_copy(dsth.at[pl.ds(0, IG)], dstb.at[0], isems[0]).wait()
    for b in range(NBUF):
        pltpu.async_copy(hp.at[srcb.at[0].at[b]], rows_v.at[b], gsems[b])

    def pair(gp, _):
        for p in range(2):
            g = gp * 2 + p
            for k in range(IG):
                b = k % 2
                pltpu.make_async_copy(hp.at[srcb.at[p].at[k]], rows_v.at[b],
                                      gsems[b]).wait()
                dvec = dstb.at[p].at[k]
                pltpu.async_copy(rows_v.at[b], acc_sh.at[dvec], ssems[b],
                                 add=True)
                pltpu.make_async_copy(rows_v.at[b], acc_sh.at[dvec],
                                      ssems[b]).wait()
                if k < IG - 2:
                    sv = srcb.at[p].at[k + 2]
                else:
                    if k == IG - 2:
                        pltpu.make_async_copy(srch.at[pl.ds(0, IG)],
                                              srcb.at[1 - p],
                                              isems[1 - p]).wait()
                        pltpu.make_async_copy(dsth.at[pl.ds(0, IG)],
                                              dstb.at[1 - p],
                                              isems[1 - p]).wait()
                    sv = srcb.at[1 - p].at[k - (IG - 2)]
                pltpu.async_copy(hp.at[sv], rows_v.at[b], gsems[b])
            goff = jnp.minimum(g + 2, NGROUP - 1) * IG
            pltpu.async_copy(srch.at[pl.ds(goff, IG)], srcb.at[p], isems[p])
            pltpu.async_copy(dsth.at[pl.ds(goff, IG)], dstb.at[p], isems[p])
        return 0

    lax.fori_loop(0, NGROUP // 2, pair, 0)
    pltpu.make_async_copy(srch.at[pl.ds(0, IG)], srcb.at[1], isems[1]).wait()
    pltpu.make_async_copy(dsth.at[pl.ds(0, IG)], dstb.at[1], isems[1]).wait()
    for b in range(NBUF):
        pltpu.make_async_copy(hp.at[srcb.at[0].at[0]], rows_v.at[b],
                              gsems[b]).wait()
    plsc.subcore_barrier()
    pltpu.sync_copy(acc_sh.at[pl.ds(t * RPT, RPT)],
                    out.at[pl.ds(t * RPT, RPT)])


@functools.cache
def _agg_call():
    return pl.kernel(
        _agg_body,
        out_type=jax.ShapeDtypeStruct((NC, NP, DH), jnp.float32),
        mesh=plsc.VectorSubcoreMesh(core_axis_name="c", subcore_axis_name="s"),
        scratch_types=(
            [pltpu.VMEM((2, IG, CH), jnp.int32),
             pltpu.VMEM((2, IG, CH), jnp.int32),
             pltpu.VMEM((NBUF, CH, DH), jnp.float32),
             pltpu.VMEM_SHARED((NP, DH), jnp.float32)]
            + [pltpu.SemaphoreType.DMA] * 6
        ),
    )


def _mm1_body(deg_ref, x_ref, w_ref, hp_ref, s_ref):
    deg = jnp.sum(deg_ref[...], axis=1) + 1.0
    s = lax.rsqrt(deg)
    h = jnp.dot(x_ref[...], w_ref[...], preferred_element_type=jnp.float32,
                precision=_HI)
    hp = h * s[:, None]
    hp_ref[0] = hp[:, :DH]
    hp_ref[1] = hp[:, DH:]
    s_ref[...] = s[:, None]


def _mm1(deg_hist, x, W1):
    return pl.pallas_call(
        _mm1_body,
        grid=(NBLK,),
        in_specs=[
            pl.BlockSpec((BLK, NW), lambda i: (i, 0)),
            pl.BlockSpec((BLK, DIN), lambda i: (i, 0)),
            pl.BlockSpec((DIN, DIN), lambda i: (0, 0)),
        ],
        out_specs=[
            pl.BlockSpec((NC, BLK, DH), lambda i: (0, i, 0)),
            pl.BlockSpec((BLK, 1), lambda i: (i, 0)),
        ],
        out_shape=[
            jax.ShapeDtypeStruct((NC, NP, DH), jnp.float32),
            jax.ShapeDtypeStruct((NP, 1), jnp.float32),
        ],
    )(deg_hist, x, W1)


def _mm2_body(m_ref, s_ref, b1_ref, w_ref, hp_ref):
    m = jnp.concatenate([m_ref[0], m_ref[1]], axis=-1)
    s = s_ref[...]
    h1 = jax.nn.relu(m * s + b1_ref[...])
    h2 = jnp.dot(h1, w_ref[...], preferred_element_type=jnp.float32,
                 precision=_HI) * s
    hp_ref[0] = h2[:, :DH]
    hp_ref[1] = h2[:, DH:]


def _mm2(M1, s, b1, W2):
    return pl.pallas_call(
        _mm2_body,
        grid=(NBLK,),
        in_specs=[
            pl.BlockSpec((NC, BLK, DH), lambda i: (0, i, 0)),
            pl.BlockSpec((BLK, 1), lambda i: (i, 0)),
            pl.BlockSpec((1, DIN), lambda i: (0, 0)),
            pl.BlockSpec((DIN, DIN), lambda i: (0, 0)),
        ],
        out_specs=pl.BlockSpec((NC, BLK, DH), lambda i: (0, i, 0)),
        out_shape=jax.ShapeDtypeStruct((NC, NP, DH), jnp.float32),
    )(M1, s, b1, W2)


def _pool_body(m_ref, s_ref, b2_ref, batch_ref, w3_ref, b3_ref, out_ref,
               acc_ref, cnt_ref):
    i = pl.program_id(0)

    @pl.when(i == 0)
    def _():
        acc_ref[...] = jnp.zeros_like(acc_ref)
        cnt_ref[...] = jnp.zeros_like(cnt_ref)

    m = jnp.concatenate([m_ref[0], m_ref[1]], axis=-1)
    b = batch_ref[0, 0, :]
    h2 = m * s_ref[...] + b2_ref[...]
    row = lax.broadcasted_iota(jnp.int32, (BLK, DIN), 0) + i * BLK
    h2 = jnp.where(row < NN, h2, 0.0)
    gid = lax.broadcasted_iota(jnp.int32, (NGRAPH, BLK), 0)
    onehot = (gid == b[None, :]).astype(jnp.float32)
    acc_ref[...] += jnp.dot(onehot, h2, preferred_element_type=jnp.float32,
                            precision=_HI)
    cnt_ref[...] += jnp.broadcast_to(
        jnp.sum(onehot, axis=1, keepdims=True), (NGRAPH, DH))

    @pl.when(i == NBLK - 1)
    def _():
        g = acc_ref[...] / jnp.maximum(cnt_ref[:, 0:1], 1.0)
        out_ref[...] = jnp.dot(g, w3_ref[...], preferred_element_type=jnp.float32,
                               precision=_HI) + b3_ref[...]


def _pool(M2, s, b2, batch3, W3, b3):
    return pl.pallas_call(
        _pool_body,
        grid=(NBLK,),
        in_specs=[
            pl.BlockSpec((NC, BLK, DH), lambda i: (0, i, 0)),
            pl.BlockSpec((BLK, 1), lambda i: (i, 0)),
            pl.BlockSpec((1, DIN), lambda i: (0, 0)),
            pl.BlockSpec((1, 1, BLK), lambda i: (i, 0, 0)),
            pl.BlockSpec((DIN, NCLS), lambda i: (0, 0)),
            pl.BlockSpec((1, NCLS), lambda i: (0, 0)),
        ],
        out_specs=pl.BlockSpec((NGRAPH, NCLS), lambda i: (0, 0)),
        out_shape=jax.ShapeDtypeStruct((NGRAPH, NCLS), jnp.float32),
        scratch_shapes=[
            pltpu.VMEM((NGRAPH, DIN), jnp.float32),
            pltpu.VMEM((NGRAPH, DH), jnp.float32),
        ],
    )(M2, s, b2, batch3, W3, b3)


@jax.jit
def kernel(x, edge_index, batch, W1, b1, W2, b2, W3, b3):
    er = edge_index.reshape(2, NS, NCHUNK, CH)
    batchp = jnp.concatenate(
        [batch, jnp.full((NP - NN,), NGRAPH, jnp.int32)])
    batch3 = batchp.reshape(NBLK, 1, BLK)
    b1r = b1.reshape(1, DIN)
    b2r = b2.reshape(1, DIN)
    b3r = b3.reshape(1, NCLS)

    deg_hist = _deg_call()(er)
    hp1, s = _mm1(deg_hist.T, x, W1)
    M1 = _agg_call()(hp1, er)
    hp2 = _mm2(M1, s, b1r, W2)
    M2 = _agg_call()(hp2, er)
    return _pool(M2, s, b2r, batch3, W3, b3r)


# speedup vs baseline: 1.1694x; 1.1694x over previous
import functools
import jax
import jax.numpy as jnp
from jax import lax
from jax.experimental import pallas as pl
from jax.experimental.pallas import tpu as pltpu
from jax.experimental.pallas import tpu_sc as plsc

NN = 10000
EE = 160000
DIN = 256
DH = 128
NGRAPH = 64
NCLS = 16
NC, NS = 2, 16
NW = NC * NS

HIST = 10240
DEG_ROWS = 40
AGG_EPT = EE // NS
CH = 125
NCHUNK = AGG_EPT // CH

NP = 10240
RPT = NP // NS
BLK = 1280
NBLK = NP // BLK

_HI = jax.lax.Precision.DEFAULT


def _deg_body(er_hbm, out_hbm, dst_v, hist_v):
    c = lax.axis_index("c")
    s = lax.axis_index("s")
    wid = s * NC + c
    t = wid // 2
    half = wid % 2
    pltpu.sync_copy(er_hbm.at[1].at[t].at[pl.ds(half * DEG_ROWS, DEG_ROWS)],
                    dst_v)

    def zero(i, _):
        hist_v[pl.ds(i * 16, 16)] = jnp.zeros((16,), jnp.float32)
        return 0

    lax.fori_loop(0, HIST // 16, zero, 0)
    ones = jnp.ones((16,), jnp.float32)
    tail = lax.broadcasted_iota(jnp.int32, (16,), 0) >= 3

    def row(r, _):
        for q in range(7):
            idx = dst_v[r, pl.ds(q * 16, 16)]
            plsc.addupdate_scatter(hist_v, [idx], ones)
        idx = dst_v[r, pl.ds(CH - 16, 16)]
        plsc.addupdate_scatter(hist_v, [idx], ones, mask=tail)
        return 0

    lax.fori_loop(0, DEG_ROWS, row, 0)
    pltpu.sync_copy(hist_v, out_hbm.at[wid])


@functools.cache
def _deg_call():
    return pl.kernel(
        _deg_body,
        out_type=jax.ShapeDtypeStruct((NW, HIST), jnp.float32),
        mesh=plsc.VectorSubcoreMesh(core_axis_name="c", subcore_axis_name="s"),
        scratch_types=[
            pltpu.VMEM((DEG_ROWS, CH), jnp.int32),
            pltpu.VMEM((HIST,), jnp.float32),
        ],
        compiler_params=pltpu.CompilerParams(needs_layout_passes=False),
    )


NBUF = 2
IG = 8
NGROUP = NCHUNK // IG


def _agg_body(hp_hbm, er_hbm, out_hbm, srcb, dstb, rows_v, acc_sh,
              *sems):
    gsems, ssems, isems = sems[0:2], sems[2:4], sems[4:6]
    c = lax.axis_index("c")
    t = lax.axis_index("s")
    hp = hp_hbm.at[c]
    out = out_hbm.at[c]
    srch = er_hbm.at[0].at[t]
    dsth = er_hbm.at[1].at[t]
    pltpu.sync_copy(hp.at[pl.ds(t * RPT, RPT)],
                    acc_sh.at[pl.ds(t * RPT, RPT)])
    for p in range(2):
        pltpu.async_copy(srch.at[pl.ds(p * IG, IG)], srcb.at[p], isems[p])
        pltpu.async_copy(dsth.at[pl.ds(p * IG, IG)], dstb.at[p], isems[p])
    plsc.subcore_barrier()
    pltpu.make_async_copy(srch.at[pl.ds(0, IG)], srcb.at[0], isems[0]).wait()
    pltpu.make_async_copy(dsth.at[pl.ds(0, IG)], dstb.at[0], isems[0]).wait()
    for b in range(NBUF):
        pltpu.async_copy(hp.at[srcb.at[0].at[b]], rows_v.at[b], gsems[b])

    def pair(gp, _):
        for p in range(2):
            g = gp * 2 + p
            for k in range(IG):
                b = k % 2
                pltpu.make_async_copy(hp.at[srcb.at[p].at[k]], rows_v.at[b],
                                      gsems[b]).wait()
                dvec = dstb.at[p].at[k]
                pltpu.async_copy(rows_v.at[b], acc_sh.at[dvec], ssems[b],
                                 add=True)
                pltpu.make_async_copy(rows_v.at[b], acc_sh.at[dvec],
                                      ssems[b]).wait()
                if k < IG - 2:
                    sv = srcb.at[p].at[k + 2]
                else:
                    if k == IG - 2:
                        pltpu.make_async_copy(srch.at[pl.ds(0, IG)],
                                              srcb.at[1 - p],
                                              isems[1 - p]).wait()
                        pltpu.make_async_copy(dsth.at[pl.ds(0, IG)],
                                              dstb.at[1 - p],
                                              isems[1 - p]).wait()
                    sv = srcb.at[1 - p].at[k - (IG - 2)]
                pltpu.async_copy(hp.at[sv], rows_v.at[b], gsems[b])
            goff = jnp.minimum(g + 2, NGROUP - 1) * IG
            pltpu.async_copy(srch.at[pl.ds(goff, IG)], srcb.at[p], isems[p])
            pltpu.async_copy(dsth.at[pl.ds(goff, IG)], dstb.at[p], isems[p])
        return 0

    lax.fori_loop(0, NGROUP // 2, pair, 0)
    pltpu.make_async_copy(srch.at[pl.ds(0, IG)], srcb.at[1], isems[1]).wait()
    pltpu.make_async_copy(dsth.at[pl.ds(0, IG)], dstb.at[1], isems[1]).wait()
    for b in range(NBUF):
        pltpu.make_async_copy(hp.at[srcb.at[0].at[0]], rows_v.at[b],
                              gsems[b]).wait()
    plsc.subcore_barrier()
    pltpu.sync_copy(acc_sh.at[pl.ds(t * RPT, RPT)],
                    out.at[pl.ds(t * RPT, RPT)])


@functools.cache
def _agg_call():
    return pl.kernel(
        _agg_body,
        out_type=jax.ShapeDtypeStruct((NC, NP, DH), jnp.float32),
        mesh=plsc.VectorSubcoreMesh(core_axis_name="c", subcore_axis_name="s"),
        scratch_types=(
            [pltpu.VMEM((2, IG, CH), jnp.int32),
             pltpu.VMEM((2, IG, CH), jnp.int32),
             pltpu.VMEM((NBUF, CH, DH), jnp.float32),
             pltpu.VMEM_SHARED((NP, DH), jnp.float32)]
            + [pltpu.SemaphoreType.DMA] * 6
        ),
    )


def _mm1_body(deg_ref, x_ref, w_ref, hp_ref, s_ref):
    deg = jnp.sum(deg_ref[...], axis=0) + 1.0
    s = lax.rsqrt(deg)
    h = jnp.dot(x_ref[...], w_ref[...], preferred_element_type=jnp.float32,
                precision=_HI)
    hp = h * s[:, None]
    hp_ref[0] = hp[:, :DH]
    hp_ref[1] = hp[:, DH:]
    s_ref[...] = s[:, None]


def _mm1(deg_hist, x, W1):
    return pl.pallas_call(
        _mm1_body,
        grid=(NBLK,),
        in_specs=[
            pl.BlockSpec((NW, BLK), lambda i: (0, i)),
            pl.BlockSpec((BLK, DIN), lambda i: (i, 0)),
            pl.BlockSpec((DIN, DIN), lambda i: (0, 0)),
        ],
        out_specs=[
            pl.BlockSpec((NC, BLK, DH), lambda i: (0, i, 0)),
            pl.BlockSpec((BLK, 1), lambda i: (i, 0)),
        ],
        out_shape=[
            jax.ShapeDtypeStruct((NC, NP, DH), jnp.float32),
            jax.ShapeDtypeStruct((NP, 1), jnp.float32),
        ],
    )(deg_hist, x, W1)


def _mm2_body(m_ref, s_ref, b1_ref, w_ref, hp_ref):
    m = jnp.concatenate([m_ref[0], m_ref[1]], axis=-1)
    s = s_ref[...]
    h1 = jax.nn.relu(m * s + b1_ref[...])
    h2 = jnp.dot(h1, w_ref[...], preferred_element_type=jnp.float32,
                 precision=_HI) * s
    hp_ref[0] = h2[:, :DH]
    hp_ref[1] = h2[:, DH:]


def _mm2(M1, s, b1, W2):
    return pl.pallas_call(
        _mm2_body,
        grid=(NBLK,),
        in_specs=[
            pl.BlockSpec((NC, BLK, DH), lambda i: (0, i, 0)),
            pl.BlockSpec((BLK, 1), lambda i: (i, 0)),
            pl.BlockSpec((1, DIN), lambda i: (0, 0)),
            pl.BlockSpec((DIN, DIN), lambda i: (0, 0)),
        ],
        out_specs=pl.BlockSpec((NC, BLK, DH), lambda i: (0, i, 0)),
        out_shape=jax.ShapeDtypeStruct((NC, NP, DH), jnp.float32),
    )(M1, s, b1, W2)


def _pool_body(m_ref, s_ref, b2_ref, batch_ref, w3_ref, b3_ref, out_ref,
               acc_ref, cnt_ref):
    i = pl.program_id(0)

    @pl.when(i == 0)
    def _():
        acc_ref[...] = jnp.zeros_like(acc_ref)
        cnt_ref[...] = jnp.zeros_like(cnt_ref)

    m = jnp.concatenate([m_ref[0], m_ref[1]], axis=-1)
    b = batch_ref[0, 0, :]
    h2 = m * s_ref[...] + b2_ref[...]
    row = lax.broadcasted_iota(jnp.int32, (BLK, DIN), 0) + i * BLK
    h2 = jnp.where(row < NN, h2, 0.0)
    gid = lax.broadcasted_iota(jnp.int32, (NGRAPH, BLK), 0)
    onehot = (gid == b[None, :]).astype(jnp.float32)
    acc_ref[...] += jnp.dot(onehot, h2, preferred_element_type=jnp.float32,
                            precision=_HI)
    cnt_ref[...] += jnp.broadcast_to(
        jnp.sum(onehot, axis=1, keepdims=True), (NGRAPH, DH))

    @pl.when(i == NBLK - 1)
    def _():
        g = acc_ref[...] / jnp.maximum(cnt_ref[:, 0:1], 1.0)
        out_ref[...] = jnp.dot(g, w3_ref[...], preferred_element_type=jnp.float32,
                               precision=_HI) + b3_ref[...]


def _pool(M2, s, b2, batch3, W3, b3):
    return pl.pallas_call(
        _pool_body,
        grid=(NBLK,),
        in_specs=[
            pl.BlockSpec((NC, BLK, DH), lambda i: (0, i, 0)),
            pl.BlockSpec((BLK, 1), lambda i: (i, 0)),
            pl.BlockSpec((1, DIN), lambda i: (0, 0)),
            pl.BlockSpec((1, 1, BLK), lambda i: (i, 0, 0)),
            pl.BlockSpec((DIN, NCLS), lambda i: (0, 0)),
            pl.BlockSpec((1, NCLS), lambda i: (0, 0)),
        ],
        out_specs=pl.BlockSpec((NGRAPH, NCLS), lambda i: (0, 0)),
        out_shape=jax.ShapeDtypeStruct((NGRAPH, NCLS), jnp.float32),
        scratch_shapes=[
            pltpu.VMEM((NGRAPH, DIN), jnp.float32),
            pltpu.VMEM((NGRAPH, DH), jnp.float32),
        ],
    )(M2, s, b2, batch3, W3, b3)


@jax.jit
def kernel(x, edge_index, batch, W1, b1, W2, b2, W3, b3):
    er = edge_index.reshape(2, NS, NCHUNK, CH)
    batchp = jnp.concatenate(
        [batch, jnp.full((NP - NN,), NGRAPH, jnp.int32)])
    batch3 = batchp.reshape(NBLK, 1, BLK)
    b1r = b1.reshape(1, DIN)
    b2r = b2.reshape(1, DIN)
    b3r = b3.reshape(1, NCLS)

    deg_hist = _deg_call()(er)
    hp1, s = _mm1(deg_hist, x, W1)
    M1 = _agg_call()(hp1, er)
    hp2 = _mm2(M1, s, b1r, W2)
    M2 = _agg_call()(hp2, er)
    return _pool(M2, s, b2r, batch3, W3, b3r)
